# baseline (speedup 1.0000x reference)
import functools

import jax
import jax.numpy as jnp
from jax import lax
from jax.experimental import pallas as pl
from jax.experimental.pallas import tpu as pltpu
from jax.experimental.pallas import tpu_sc as plsc

NC, NS = 2, 16
NW = NC * NS


def _make_sc_kernel(batch, seq_len, dim, dtype):
    rows_per_w = seq_len // NW
    chunk = min(32, rows_per_w)
    n_chunks = rows_per_w // chunk
    n_buf = min(4, n_chunks)
    mesh = plsc.VectorSubcoreMesh(core_axis_name="c", subcore_axis_name="s")

    @functools.partial(
        pl.kernel,
        mesh=mesh,
        out_type=jax.ShapeDtypeStruct((batch, seq_len, dim), dtype),
    scratch_types=(
        [pltpu.VMEM((chunk, dim), dtype)] * n_buf
        + [pltpu.SemaphoreType.DMA, pltpu.SemaphoreType.DMA]
    ),
    )
    def sc_kernel(table_hbm, out_hbm, *rest):
        bufs, (gsem, ssem) = list(rest[:n_buf]), rest[n_buf:]
        wid = lax.axis_index("s") * NC + lax.axis_index("c")
        base = wid * rows_per_w
        gathers = [None] * n_chunks
        scatters = [None] * n_chunks
        for c in range(n_buf - 1):
            gathers[c] = pltpu.async_copy(
                table_hbm.at[pl.ds(base + c * chunk, chunk)], bufs[c], gsem)
        for c in range(n_chunks):
            off = base + c * chunk
            gathers[c].wait()
            nxt = c + n_buf - 1
            if nxt < n_chunks:
                if nxt - n_buf >= 0:
                    for cp in scatters[nxt - n_buf]:
                        cp.wait()
                gathers[nxt] = pltpu.async_copy(
                    table_hbm.at[pl.ds(base + nxt * chunk, chunk)],
                    bufs[nxt % n_buf], gsem)
            scatters[c] = [
                pltpu.async_copy(bufs[c % n_buf],
                                 out_hbm.at[(b + wid) % batch,
                                            pl.ds(off, chunk)],
                                 ssem)
                for b in range(batch)
            ]
        for c in range(max(0, n_chunks - n_buf), n_chunks):
            for cp in scatters[c]:
                cp.wait()

    return sc_kernel


def _tc_copy(batch, seq_len, dim, table):
    blk = 512

    def body(t_ref, o_ref):
        o_ref[...] = t_ref[...][None]

    return pl.pallas_call(
        body,
        grid=(seq_len // blk, batch),
        in_specs=[pl.BlockSpec((blk, dim), lambda i, b: (i, 0))],
        out_specs=pl.BlockSpec((1, blk, dim), lambda i, b: (b, i, 0)),
        out_shape=jax.ShapeDtypeStruct((batch, seq_len, dim), table.dtype),
    )(table)


def kernel(sequence, table):
    batch, seq_len = sequence.shape
    dim = table.shape[1]
    return _make_sc_kernel(batch, seq_len, dim, table.dtype)(table)


# speedup vs baseline: 1.0208x; 1.0208x over previous
import functools

import jax
import jax.numpy as jnp
from jax import lax
from jax.experimental import pallas as pl
from jax.experimental.pallas import tpu as pltpu
from jax.experimental.pallas import tpu_sc as plsc

NC, NS = 2, 16
NW = NC * NS


def _make_sc_kernel(batch, seq_len, dim, dtype):
    rows_per_w = seq_len // NW
    chunk = min(64, rows_per_w)
    n_chunks = rows_per_w // chunk
    n_buf = min(2, n_chunks)
    mesh = plsc.VectorSubcoreMesh(core_axis_name="c", subcore_axis_name="s")

    @functools.partial(
        pl.kernel,
        mesh=mesh,
        out_type=jax.ShapeDtypeStruct((batch, seq_len, dim), dtype),
    scratch_types=(
        [pltpu.VMEM((chunk, dim), dtype)] * n_buf
        + [pltpu.SemaphoreType.DMA, pltpu.SemaphoreType.DMA]
    ),
    )
    def sc_kernel(table_hbm, out_hbm, *rest):
        bufs, (gsem, ssem) = list(rest[:n_buf]), rest[n_buf:]
        wid = lax.axis_index("s") * NC + lax.axis_index("c")
        base = wid * rows_per_w
        gathers = [None] * n_chunks
        scatters = [None] * n_chunks
        for c in range(n_buf - 1):
            gathers[c] = pltpu.async_copy(
                table_hbm.at[pl.ds(base + c * chunk, chunk)], bufs[c], gsem)
        for c in range(n_chunks):
            off = base + c * chunk
            gathers[c].wait()
            nxt = c + n_buf - 1
            if nxt < n_chunks:
                if nxt - n_buf >= 0:
                    for cp in scatters[nxt - n_buf]:
                        cp.wait()
                gathers[nxt] = pltpu.async_copy(
                    table_hbm.at[pl.ds(base + nxt * chunk, chunk)],
                    bufs[nxt % n_buf], gsem)
            scatters[c] = [
                pltpu.async_copy(bufs[c % n_buf],
                                 out_hbm.at[(b + wid) % batch,
                                            pl.ds(off, chunk)],
                                 ssem)
                for b in range(batch)
            ]
        for c in range(max(0, n_chunks - n_buf), n_chunks):
            for cp in scatters[c]:
                cp.wait()

    return sc_kernel


def _tc_copy(batch, seq_len, dim, table):
    blk = 512

    def body(t_ref, o_ref):
        o_ref[...] = t_ref[...][None]

    return pl.pallas_call(
        body,
        grid=(seq_len // blk, batch),
        in_specs=[pl.BlockSpec((blk, dim), lambda i, b: (i, 0))],
        out_specs=pl.BlockSpec((1, blk, dim), lambda i, b: (b, i, 0)),
        out_shape=jax.ShapeDtypeStruct((batch, seq_len, dim), table.dtype),
    )(table)


def kernel(sequence, table):
    batch, seq_len = sequence.shape
    dim = table.shape[1]
    return _make_sc_kernel(batch, seq_len, dim, table.dtype)(table)


# speedup vs baseline: 1.0246x; 1.0038x over previous
import functools

import jax
from jax import lax
from jax.experimental import pallas as pl
from jax.experimental.pallas import tpu as pltpu
from jax.experimental.pallas import tpu_sc as plsc

NC, NS = 2, 16
NW = NC * NS


def _make_sc_kernel(batch, seq_len, dim, dtype):
    rows_per_w = seq_len // NW
    chunk = min(64, rows_per_w)
    n_chunks = rows_per_w // chunk
    n_buf = 2 if n_chunks >= 2 else 1
    mesh = plsc.VectorSubcoreMesh(core_axis_name="c", subcore_axis_name="s")

    @functools.partial(
        pl.kernel,
        mesh=mesh,
        out_type=jax.ShapeDtypeStruct((batch, seq_len, dim), dtype),
        scratch_types=(
            [pltpu.VMEM((chunk, dim), dtype)] * n_buf
            + [pltpu.SemaphoreType.DMA, pltpu.SemaphoreType.DMA]
        ),
    )
    def sc_kernel(table_hbm, out_hbm, *rest):
        bufs, (gsem, ssem) = list(rest[:n_buf]), rest[n_buf:]
        wid = lax.axis_index("s") * NC + lax.axis_index("c")
        base = wid * rows_per_w

        if n_buf == 1:
            for c in range(n_chunks):
                off = base + c * chunk
                pltpu.sync_copy(table_hbm.at[pl.ds(off, chunk)], bufs[0])
                copies = [
                    pltpu.async_copy(bufs[0], out_hbm.at[b, pl.ds(off, chunk)],
                                     ssem)
                    for b in range(batch)
                ]
                for cp in copies:
                    cp.wait()
            return

        gathers = [None] * n_chunks
        scatters = [None] * n_chunks
        for c in range(n_buf - 1):
            gathers[c] = pltpu.async_copy(
                table_hbm.at[pl.ds(base + c * chunk, chunk)], bufs[c], gsem)
        for c in range(n_chunks):
            off = base + c * chunk
            gathers[c].wait()
            nxt = c + n_buf - 1
            if nxt < n_chunks:
                if nxt - n_buf >= 0:
                    for cp in scatters[nxt - n_buf]:
                        cp.wait()
                gathers[nxt] = pltpu.async_copy(
                    table_hbm.at[pl.ds(base + nxt * chunk, chunk)],
                    bufs[nxt % n_buf], gsem)
            scatters[c] = [
                pltpu.async_copy(bufs[c % n_buf],
                                 out_hbm.at[b, pl.ds(off, chunk)], ssem)
                for b in range(batch)
            ]
        for c in range(max(0, n_chunks - n_buf), n_chunks):
            for cp in scatters[c]:
                cp.wait()

    return sc_kernel


def kernel(sequence, table):
    batch, seq_len = sequence.shape
    dim = table.shape[1]
    return _make_sc_kernel(batch, seq_len, dim, table.dtype)(table)
